# baseline (speedup 1.0000x reference)
import jax
import jax.numpy as jnp
from jax import lax
from jax.experimental import pallas as pl
from jax.experimental.pallas import tpu as pltpu
from jax.experimental.pallas import tpu_sc as plsc

KB = 100
KBP = 128
CC = 25
CCP = 32
TOPK = 10
NEG = -1e9

_INTERPRET = False

_NW = 32
_CH = 64
_K = 8
_GR = _CH * _K


def _sc_gather_body(wn_table, nell_table, wn_idx, nell_idx, wn_out, nell_out,
                    idx_v, rows_v, gsem, wsem):
    wid = lax.axis_index("s") * 2 + lax.axis_index("c")
    n_rows = wn_idx.shape[0] // _NW
    nch = n_rows // _CH
    base = pl.multiple_of(wid * n_rows, n_rows)

    pltpu.sync_copy(wn_idx.at[pl.ds(base, n_rows)], idx_v.at[pl.ds(0, n_rows)])
    pltpu.sync_copy(nell_idx.at[pl.ds(base, n_rows)],
                    idx_v.at[pl.ds(n_rows, n_rows)])

    def phase(table, out, idx_base):
        def group(g, carry):
            goff = pl.multiple_of(g * _GR, _GR)
            for b in range(_K):
                off = idx_base + goff + b * _CH
                pltpu.async_copy(table.at[idx_v.at[pl.ds(off, _CH)]],
                                 rows_v.at[pl.ds(b * _CH, _CH)], gsem)
            pltpu.make_async_copy(table.at[pl.ds(0, _GR)], rows_v, gsem).wait()
            pltpu.async_copy(rows_v, out.at[pl.ds(base + goff, _GR)],
                             wsem).wait()
            return carry

        lax.fori_loop(0, n_rows // _GR, group, 0, unroll=False)

    phase(wn_table, wn_out, 0)
    phase(nell_table, nell_out, n_rows)


def _sc_gather(wn_table, nell_table, wn_idx, nell_idx):
    n = wn_idx.size
    n_rows = n // _NW
    mesh = plsc.VectorSubcoreMesh(core_axis_name="c", subcore_axis_name="s")
    f = pl.kernel(
        _sc_gather_body,
        out_type=[
            jax.ShapeDtypeStruct((n, KBP), jnp.float32),
            jax.ShapeDtypeStruct((n, KBP), jnp.float32),
        ],
        mesh=mesh,
        compiler_params=pltpu.CompilerParams(use_tc_tiling_on_sc=False),
        scratch_types=[
            pltpu.VMEM((2 * n_rows,), jnp.int32),
            pltpu.VMEM((_GR, KBP), jnp.float32),
            pltpu.SemaphoreType.DMA,
            pltpu.SemaphoreType.DMA,
        ],
    )
    return f(wn_table, nell_table, wn_idx.reshape(n), nell_idx.reshape(n))


def _dense_body(slot_ref, intent_ref, wn_cv_ref, nell_cv_ref, wn_idx_ref,
                nell_idx_ref, km_r_ref, km_c_ref, wk_ref, wc_ref, gamma_ref,
                beta_ref, pe_ref, know_ref, ctx_ref):
    D = slot_ref.shape[-1]
    slot = slot_ref[0]
    intent = intent_ref[0]
    wk = wk_ref[...]
    wc = wc_ref[...]
    q = (jnp.dot(slot, wk[:D], preferred_element_type=jnp.float32)
         + jnp.dot(intent, wk[D:], preferred_element_type=jnp.float32))
    q2 = (jnp.dot(slot, wc[:D], preferred_element_type=jnp.float32)
          + jnp.dot(intent, wc[D:], preferred_element_type=jnp.float32))

    wn_cv = wn_cv_ref[0]
    nell_cv = nell_cv_ref[0]
    s_wn = jnp.sum(q[:, None, :] * wn_cv, axis=-1)
    s_nell = jnp.sum(q[:, None, :] * nell_cv, axis=-1)
    scores = jnp.concatenate([s_wn, s_nell], axis=1)
    idx = jnp.concatenate([wn_idx_ref[0], nell_idx_ref[0]], axis=1)
    scores = jnp.where(idx == 0, NEG, scores)

    S = scores.shape[0]
    rem = scores
    thresh = jnp.full((S, 1), NEG, jnp.float32)
    taken = jnp.zeros((S, 1), jnp.int32)
    done = jnp.zeros((S, 1), jnp.bool_)
    for _ in range(TOPK):
        m = jnp.max(rem, axis=1, keepdims=True)
        c = jnp.sum((rem == m).astype(jnp.int32), axis=1, keepdims=True)
        new_taken = taken + c
        thresh = jnp.where(done, thresh, m)
        rem = jnp.where(jnp.logical_and(jnp.logical_not(done), rem == m),
                        -jnp.inf, rem)
        taken = jnp.where(done, taken, new_taken)
        done = jnp.logical_or(done, new_taken >= TOPK)

    masked = jnp.where(scores < thresh, NEG, scores)
    mx = jnp.max(masked, axis=1, keepdims=True)
    e = jnp.exp(masked - mx)
    attn = e / jnp.sum(e, axis=1, keepdims=True)

    know = (jnp.sum(attn[:, :CCP, None] * wn_cv, axis=1)
            + jnp.sum(attn[:, CCP:, None] * nell_cv, axis=1))

    km_c = km_c_ref[0]
    pe = pe_ref[...]
    know = know + jnp.where(km_c == 0, 0.0, pe)

    km_r = km_r_ref[0]
    s2 = lax.dot_general(q2, know, (((1,), (1,)), ((), ())),
                         preferred_element_type=jnp.float32)
    s2 = jnp.where(km_r == 0, NEG, s2)
    mx2 = jnp.max(s2, axis=1, keepdims=True)
    e2 = jnp.exp(s2 - mx2)
    a2 = e2 / jnp.sum(e2, axis=1, keepdims=True)
    ctx = jnp.dot(a2, know, preferred_element_type=jnp.float32)

    ctx = ctx[:, :KB]
    mu = jnp.mean(ctx, axis=1, keepdims=True)
    var = jnp.mean((ctx - mu) ** 2, axis=1, keepdims=True)
    ctx = gamma_ref[...] * (ctx - mu) * lax.rsqrt(var + 1e-5) + beta_ref[...]

    know_ref[0] = know[:, :KB]
    ctx_ref[0] = ctx


def _dense(slot, intent, wn_cv, nell_cv, wn_idx, nell_idx, km, wk, wc,
           gamma, beta, pe):
    B, S, D = slot.shape
    km_r = km.reshape(B, 1, S)
    km_c = km.reshape(B, S, 1)
    bspec = lambda shp: pl.BlockSpec((1,) + shp, lambda b: (b,) + (0,) * len(shp))
    full = lambda shp: pl.BlockSpec(shp, lambda b: (0,) * len(shp))
    return pl.pallas_call(
        _dense_body,
        grid=(B,),
        in_specs=[
            bspec((S, D)), bspec((S, D)),
            bspec((S, CCP, KBP)), bspec((S, CCP, KBP)),
            bspec((S, CCP)), bspec((S, CCP)),
            bspec((1, S)), bspec((S, 1)),
            full((2 * D, KBP)), full((2 * D, KBP)),
            full((1, KB)), full((1, KB)), full((S, KBP)),
        ],
        out_specs=[bspec((S, KB)), bspec((S, KB))],
        out_shape=[
            jax.ShapeDtypeStruct((B, S, KB), jnp.float32),
            jax.ShapeDtypeStruct((B, S, KB), jnp.float32),
        ],
        interpret=_INTERPRET,
    )(slot, intent, wn_cv, nell_cv, wn_idx, nell_idx, km_r, km_c, wk, wc,
      gamma.reshape(1, KB), beta.reshape(1, KB), pe)


def kernel(intent_features, slot_features, attention_mask, wn_synset_indexes,
           wn_synset_lengths, nell_entity_indexes, nell_entity_lengths,
           wn_table, nell_table, W_k, W_c, gamma, beta, pos_embed):
    B, S, D = slot_features.shape
    padf = ((0, 0), (0, KBP - KB))
    wn_t = jnp.pad(wn_table, padf)
    nell_t = jnp.pad(nell_table, padf)
    padc = ((0, 0), (0, 0), (0, CCP - CC))
    wn_idx = jnp.pad(wn_synset_indexes.astype(jnp.int32), padc)
    nell_idx = jnp.pad(nell_entity_indexes.astype(jnp.int32), padc)
    wn_cv, nell_cv = _sc_gather(wn_t, nell_t, wn_idx, nell_idx)
    wn_cv = wn_cv.reshape(B, S, CCP, KBP)
    nell_cv = nell_cv.reshape(B, S, CCP, KBP)
    km = (wn_synset_lengths + nell_entity_lengths).astype(jnp.int32)
    know, ctx = _dense(slot_features, intent_features, wn_cv, nell_cv,
                       wn_idx, nell_idx, km,
                       jnp.pad(W_k, padf), jnp.pad(W_c, padf),
                       gamma, beta, jnp.pad(pos_embed, padf))
    return (know, ctx)


# speedup vs baseline: 1.7124x; 1.7124x over previous
import jax
import jax.numpy as jnp
from jax import lax
from jax.experimental import pallas as pl
from jax.experimental.pallas import tpu as pltpu
from jax.experimental.pallas import tpu_sc as plsc

KB = 100
KBP = 128
CC = 25
CCP = 32
TOPK = 10
NEG = -1e9

_INTERPRET = False

_NW = 32
_CH = 128


def _sc_gather_body(wn_table, nell_table, wn_idx, nell_idx, wn_out, nell_out,
                    idx_v, rows_v, gsem, wsem):
    wid = lax.axis_index("s") * 2 + lax.axis_index("c")
    n_rows = wn_idx.shape[0] // _NW
    nch = n_rows // _CH
    base = pl.multiple_of(wid * n_rows, n_rows)

    pltpu.sync_copy(wn_idx.at[pl.ds(base, n_rows)], idx_v.at[pl.ds(0, n_rows)])
    pltpu.sync_copy(nell_idx.at[pl.ds(base, n_rows)],
                    idx_v.at[pl.ds(n_rows, n_rows)])

    def chunk(table, out, idx_off, off, buf, drain_prev):
        if drain_prev:
            pltpu.make_async_copy(rows_v.at[buf], out.at[pl.ds(0, _CH)],
                                  wsem).wait()
        pltpu.async_copy(table.at[idx_v.at[pl.ds(idx_off, _CH)]],
                         rows_v.at[buf], gsem).wait()
        pltpu.async_copy(rows_v.at[buf], out.at[pl.ds(base + off, _CH)], wsem)

    chunk(wn_table, wn_out, 0, 0, 0, False)
    chunk(nell_table, nell_out, n_rows, 0, 1, False)

    def body(j, carry):
        off = pl.multiple_of(j * _CH, _CH)
        chunk(wn_table, wn_out, off, off, 0, True)
        chunk(nell_table, nell_out, n_rows + off, off, 1, True)
        return carry

    lax.fori_loop(1, nch, body, 0, unroll=False)
    pltpu.make_async_copy(rows_v.at[0], wn_out.at[pl.ds(0, _CH)], wsem).wait()
    pltpu.make_async_copy(rows_v.at[1], wn_out.at[pl.ds(0, _CH)], wsem).wait()


def _sc_gather(wn_table, nell_table, wn_idx, nell_idx):
    n = wn_idx.size
    n_rows = n // _NW
    mesh = plsc.VectorSubcoreMesh(core_axis_name="c", subcore_axis_name="s")
    f = pl.kernel(
        _sc_gather_body,
        out_type=[
            jax.ShapeDtypeStruct((n, KBP), jnp.float32),
            jax.ShapeDtypeStruct((n, KBP), jnp.float32),
        ],
        mesh=mesh,
        scratch_types=[
            pltpu.VMEM((2 * n_rows,), jnp.int32),
            pltpu.VMEM((2, _CH, KBP), jnp.float32),
            pltpu.SemaphoreType.DMA,
            pltpu.SemaphoreType.DMA,
        ],
    )
    return f(wn_table, nell_table, wn_idx.reshape(n), nell_idx.reshape(n))


def _pad_body(x_ref, o_ref):
    x = x_ref[...]
    o_ref[...] = jnp.concatenate(
        [x, jnp.zeros((x.shape[0], KBP - KB), jnp.float32)], axis=1)


def _pad_table(table, rows_per_block):
    v = table.shape[0]
    grid = (v // rows_per_block,)
    return pl.pallas_call(
        _pad_body,
        grid=grid,
        in_specs=[pl.BlockSpec((rows_per_block, KB), lambda i: (i, 0))],
        out_specs=pl.BlockSpec((rows_per_block, KBP), lambda i: (i, 0)),
        out_shape=jax.ShapeDtypeStruct((v, KBP), jnp.float32),
        interpret=_INTERPRET,
    )(table)


def _dense_body(slot_ref, intent_ref, wn_cv_ref, nell_cv_ref, wn_idx_ref,
                nell_idx_ref, km_r_ref, km_c_ref, wk_ref, wc_ref, gamma_ref,
                beta_ref, pe_ref, know_ref, ctx_ref):
    D = slot_ref.shape[-1]
    slot = slot_ref[0]
    intent = intent_ref[0]
    wk = wk_ref[...]
    wc = wc_ref[...]
    q = (jnp.dot(slot, wk[:D], preferred_element_type=jnp.float32)
         + jnp.dot(intent, wk[D:], preferred_element_type=jnp.float32))
    q2 = (jnp.dot(slot, wc[:D], preferred_element_type=jnp.float32)
          + jnp.dot(intent, wc[D:], preferred_element_type=jnp.float32))

    wn_cv = wn_cv_ref[0]
    nell_cv = nell_cv_ref[0]
    s_wn = jnp.sum(q[:, None, :] * wn_cv, axis=-1)
    s_nell = jnp.sum(q[:, None, :] * nell_cv, axis=-1)
    scores = jnp.concatenate([s_wn, s_nell], axis=1)
    idx = jnp.concatenate([wn_idx_ref[0], nell_idx_ref[0]], axis=1)
    scores = jnp.where(idx == 0, NEG, scores)

    S = scores.shape[0]
    rem = scores
    thresh = jnp.full((S, 1), NEG, jnp.float32)
    taken = jnp.zeros((S, 1), jnp.int32)
    done = jnp.zeros((S, 1), jnp.bool_)
    for _ in range(TOPK):
        m = jnp.max(rem, axis=1, keepdims=True)
        c = jnp.sum((rem == m).astype(jnp.int32), axis=1, keepdims=True)
        new_taken = taken + c
        thresh = jnp.where(done, thresh, m)
        rem = jnp.where(jnp.logical_and(jnp.logical_not(done), rem == m),
                        -jnp.inf, rem)
        taken = jnp.where(done, taken, new_taken)
        done = jnp.logical_or(done, new_taken >= TOPK)

    masked = jnp.where(scores < thresh, NEG, scores)
    mx = jnp.max(masked, axis=1, keepdims=True)
    e = jnp.exp(masked - mx)
    attn = e / jnp.sum(e, axis=1, keepdims=True)

    know = (jnp.sum(attn[:, :CCP, None] * wn_cv, axis=1)
            + jnp.sum(attn[:, CCP:, None] * nell_cv, axis=1))

    km_c = km_c_ref[0]
    pe = pe_ref[...]
    know = know + jnp.where(km_c == 0, 0.0, pe)

    km_r = km_r_ref[0]
    s2 = lax.dot_general(q2, know, (((1,), (1,)), ((), ())),
                         preferred_element_type=jnp.float32)
    s2 = jnp.where(km_r == 0, NEG, s2)
    mx2 = jnp.max(s2, axis=1, keepdims=True)
    e2 = jnp.exp(s2 - mx2)
    a2 = e2 / jnp.sum(e2, axis=1, keepdims=True)
    ctx = jnp.dot(a2, know, preferred_element_type=jnp.float32)

    ctx = ctx[:, :KB]
    mu = jnp.mean(ctx, axis=1, keepdims=True)
    var = jnp.mean((ctx - mu) ** 2, axis=1, keepdims=True)
    ctx = gamma_ref[...] * (ctx - mu) * lax.rsqrt(var + 1e-5) + beta_ref[...]

    know_ref[0] = know[:, :KB]
    ctx_ref[0] = ctx


def _dense(slot, intent, wn_cv, nell_cv, wn_idx, nell_idx, km, wk, wc,
           gamma, beta, pe):
    B, S, D = slot.shape
    km_r = km.reshape(B, 1, S)
    km_c = km.reshape(B, S, 1)
    bspec = lambda shp: pl.BlockSpec((1,) + shp, lambda b: (b,) + (0,) * len(shp))
    full = lambda shp: pl.BlockSpec(shp, lambda b: (0,) * len(shp))
    return pl.pallas_call(
        _dense_body,
        grid=(B,),
        in_specs=[
            bspec((S, D)), bspec((S, D)),
            bspec((S, CCP, KBP)), bspec((S, CCP, KBP)),
            bspec((S, CCP)), bspec((S, CCP)),
            bspec((1, S)), bspec((S, 1)),
            full((2 * D, KBP)), full((2 * D, KBP)),
            full((1, KB)), full((1, KB)), full((S, KBP)),
        ],
        out_specs=[bspec((S, KB)), bspec((S, KB))],
        out_shape=[
            jax.ShapeDtypeStruct((B, S, KB), jnp.float32),
            jax.ShapeDtypeStruct((B, S, KB), jnp.float32),
        ],
        interpret=_INTERPRET,
    )(slot, intent, wn_cv, nell_cv, wn_idx, nell_idx, km_r, km_c, wk, wc,
      gamma.reshape(1, KB), beta.reshape(1, KB), pe)


def kernel(intent_features, slot_features, attention_mask, wn_synset_indexes,
           wn_synset_lengths, nell_entity_indexes, nell_entity_lengths,
           wn_table, nell_table, W_k, W_c, gamma, beta, pos_embed):
    B, S, D = slot_features.shape
    padf = ((0, 0), (0, KBP - KB))
    wn_t = _pad_table(wn_table, 2000)
    nell_t = _pad_table(nell_table, 2000)
    padc = ((0, 0), (0, 0), (0, CCP - CC))
    wn_idx = jnp.pad(wn_synset_indexes.astype(jnp.int32), padc)
    nell_idx = jnp.pad(nell_entity_indexes.astype(jnp.int32), padc)
    wn_cv, nell_cv = _sc_gather(wn_t, nell_t, wn_idx, nell_idx)
    wn_cv = wn_cv.reshape(B, S, CCP, KBP)
    nell_cv = nell_cv.reshape(B, S, CCP, KBP)
    km = (wn_synset_lengths + nell_entity_lengths).astype(jnp.int32)
    know, ctx = _dense(slot_features, intent_features, wn_cv, nell_cv,
                       wn_idx, nell_idx, km,
                       jnp.pad(W_k, padf), jnp.pad(W_c, padf),
                       gamma, beta, jnp.pad(pos_embed, padf))
    return (know, ctx)
